# Initial kernel scaffold; baseline (speedup 1.0000x reference)
import functools

import jax
import jax.numpy as jnp
from jax import lax
from jax.experimental import pallas as pl
from jax.experimental.pallas import tpu as pltpu
from jax.experimental.pallas import tpu_sc as plsc

NNODE = 10000
NPAD = 10240
DIN = 480
DOUT = 256
HALF = 128
NEG = 0.2
NEDGE = 160000
EP = 172032
KB = 64
NC = 2
NS = 16
LL = 16
TILES = NC * NS
E1 = EP // TILES
B1 = E1 // KB
E2 = EP // NS
B2 = E2 // KB
RPT = NPAD // NS

_mesh = plsc.VectorSubcoreMesh(core_axis_name="c", subcore_axis_name="s")


@functools.partial(
    pl.kernel,
    out_type=jax.ShapeDtypeStruct((EP,), jnp.float32),
    mesh=_mesh,
    scratch_types=[
        pltpu.VMEM((KB,), jnp.int32),
        pltpu.VMEM((KB,), jnp.int32),
        pltpu.VMEM((KB, HALF), jnp.float32),
        pltpu.VMEM((KB, HALF), jnp.float32),
        pltpu.VMEM((KB, HALF), jnp.float32),
        pltpu.VMEM((KB, HALF), jnp.float32),
        pltpu.VMEM((KB, LL), jnp.float32),
        pltpu.VMEM((KB,), jnp.float32),
        pltpu.VMEM((DOUT,), jnp.float32),
    ],
)
def _edge_logits(src_hbm, dst_hbm, xl0, xl1, xr0, xr1, att_hbm, w_hbm,
                 srci, dsti, a0, a1, b0, b1, tbuf, wv, attv):
    c = lax.axis_index("c")
    s = lax.axis_index("s")
    wid = s * NC + c
    pltpu.sync_copy(att_hbm, attv)
    att_chunks = [attv[pl.ds(i * LL, LL)] for i in range(DOUT // LL)]
    iota = lax.iota(jnp.int32, LL)

    @pl.loop(0, B1)
    def _blk(blk):
        base = wid * E1 + blk * KB
        pltpu.sync_copy(src_hbm.at[pl.ds(base, KB)], srci)
        pltpu.sync_copy(dst_hbm.at[pl.ds(base, KB)], dsti)
        pltpu.sync_copy(xl0.at[srci], a0)
        pltpu.sync_copy(xl1.at[srci], a1)
        pltpu.sync_copy(xr0.at[dsti], b0)
        pltpu.sync_copy(xr1.at[dsti], b1)

        @pl.loop(0, KB)
        def _edge(e):
            acc = jnp.zeros((LL,), jnp.float32)
            for half, (ab, bb) in enumerate(((a0, b0), (a1, b1))):
                for q in range(HALF // LL):
                    sl = pl.ds(q * LL, LL)
                    sv = ab[e, sl] + bb[e, sl]
                    m = jnp.maximum(sv, NEG * sv)
                    acc = acc + att_chunks[half * (HALF // LL) + q] * m
            tbuf[e, :] = acc

        for g in range(KB // LL):
            rows = iota + (g * LL)
            tot = jnp.zeros((LL,), jnp.float32)
            for cc in range(LL):
                cols = jnp.full((LL,), cc, jnp.int32)
                tot = tot + plsc.load_gather(tbuf, [rows, cols])
            wv[pl.ds(g * LL, LL)] = jnp.exp(tot)

        pltpu.sync_copy(wv, w_hbm.at[pl.ds(base, KB)])


@functools.partial(
    pl.kernel,
    out_type=[
        jax.ShapeDtypeStruct((NPAD, HALF), jnp.float32),
        jax.ShapeDtypeStruct((NPAD, HALF), jnp.float32),
        jax.ShapeDtypeStruct((NPAD, LL), jnp.float32),
    ],
    mesh=_mesh,
    scratch_types=[
        pltpu.VMEM((KB,), jnp.int32),
        pltpu.VMEM((KB,), jnp.int32),
        pltpu.VMEM((KB,), jnp.float32),
        pltpu.VMEM((KB, HALF), jnp.float32),
        pltpu.VMEM((KB, LL), jnp.float32),
        pltpu.VMEM((KB, HALF), jnp.float32),
        pltpu.VMEM_SHARED((NPAD, HALF), jnp.float32),
        pltpu.VMEM_SHARED((NPAD, LL), jnp.float32),
    ],
)
def _aggregate(src_hbm, dst_hbm, w_hbm, xl0, xl1,
               acc0_hbm, acc1_hbm, accw_hbm,
               srci, dsti, wv, rows, wcol, zb, accv, accw):
    c = lax.axis_index("c")
    s = lax.axis_index("s")
    iota = lax.iota(jnp.int32, LL)
    rbase = s * RPT

    @pl.loop(0, KB)
    def _z(i):
        for q in range(HALF // LL):
            zb[i, pl.ds(q * LL, LL)] = jnp.zeros((LL,), jnp.float32)
        wcol[i, :] = jnp.zeros((LL,), jnp.float32)

    @pl.loop(0, RPT // KB)
    def _zz(i):
        pltpu.sync_copy(zb, accv.at[pl.ds(rbase + i * KB, KB)])
        pltpu.sync_copy(wcol, accw.at[pl.ds(rbase + i * KB, KB)])

    plsc.subcore_barrier()

    @pl.loop(0, B2)
    def _blk(blk):
        base = s * E2 + blk * KB
        pltpu.sync_copy(src_hbm.at[pl.ds(base, KB)], srci)
        pltpu.sync_copy(dst_hbm.at[pl.ds(base, KB)], dsti)
        pltpu.sync_copy(w_hbm.at[pl.ds(base, KB)], wv)

        @pl.when(c == 0)
        def _():
            pltpu.sync_copy(xl0.at[srci], rows)

        @pl.when(c == 1)
        def _():
            pltpu.sync_copy(xl1.at[srci], rows)

        @pl.loop(0, KB)
        def _edge(e):
            espl = jnp.zeros((LL,), jnp.int32) + e
            wspl = plsc.load_gather(wv, [espl])
            for q in range(HALF // LL):
                sl = pl.ds(q * LL, LL)
                rows[e, sl] = rows[e, sl] * wspl

        for g in range(KB // LL):
            rid = iota + g * LL
            w16 = wv[pl.ds(g * LL, LL)]
            plsc.store_scatter(wcol, [rid, jnp.zeros((LL,), jnp.int32)], w16)

        pltpu.sync_copy(rows, accv.at[dsti], add=True)
        pltpu.sync_copy(wcol, accw.at[dsti], add=True)

    plsc.subcore_barrier()

    @pl.when(c == 0)
    def _flush0():
        @pl.loop(0, RPT // KB)
        def _f(i):
            sl = pl.ds(rbase + i * KB, KB)
            pltpu.sync_copy(accv.at[sl], acc0_hbm.at[sl])
            pltpu.sync_copy(accw.at[sl], accw_hbm.at[sl])

    @pl.when(c == 1)
    def _flush1():
        @pl.loop(0, RPT // KB)
        def _f(i):
            sl = pl.ds(rbase + i * KB, KB)
            pltpu.sync_copy(accv.at[sl], acc1_hbm.at[sl])


def _matmul_call(xp, W_l, W_r):
    bm = 640

    def mm_kernel(x_ref, wl_ref, wr_ref, o0, o1, o2, o3):
        xb = x_ref[...]
        xl = jnp.dot(xb, wl_ref[...], preferred_element_type=jnp.float32,
                     precision=lax.Precision.HIGHEST)
        xr = jnp.dot(xb, wr_ref[...], preferred_element_type=jnp.float32,
                     precision=lax.Precision.HIGHEST)
        o0[...] = xl[:, :HALF]
        o1[...] = xl[:, HALF:]
        o2[...] = xr[:, :HALF]
        o3[...] = xr[:, HALF:]

    return pl.pallas_call(
        mm_kernel,
        grid=(NPAD // bm,),
        in_specs=[
            pl.BlockSpec((bm, DIN), lambda i: (i, 0)),
            pl.BlockSpec((DIN, DOUT), lambda i: (0, 0)),
            pl.BlockSpec((DIN, DOUT), lambda i: (0, 0)),
        ],
        out_specs=[pl.BlockSpec((bm, HALF), lambda i: (i, 0))] * 4,
        out_shape=[jax.ShapeDtypeStruct((NPAD, HALF), jnp.float32)] * 4,
    )(xp, W_l, W_r)


def _finalize_call(acc0, acc1, accw, bias2d):
    bm = 400

    def fin_kernel(a0, a1, aw, b_ref, o_ref):
        d = aw[...][:, 0:1]
        o_ref[...] = jnp.concatenate([a0[...], a1[...]], axis=1) / d + b_ref[...]

    return pl.pallas_call(
        fin_kernel,
        grid=(NNODE // bm,),
        in_specs=[
            pl.BlockSpec((bm, HALF), lambda i: (i, 0)),
            pl.BlockSpec((bm, HALF), lambda i: (i, 0)),
            pl.BlockSpec((bm, LL), lambda i: (i, 0)),
            pl.BlockSpec((1, DOUT), lambda i: (0, 0)),
        ],
        out_specs=pl.BlockSpec((bm, DOUT), lambda i: (i, 0)),
        out_shape=jax.ShapeDtypeStruct((NNODE, DOUT), jnp.float32),
    )(acc0, acc1, accw, bias2d)


def kernel(x, edge_index, W_l, W_r, att, bias):
    loop = jnp.arange(NNODE, dtype=jnp.int32)
    pad = EP - NEDGE - NNODE
    src = jnp.concatenate(
        [edge_index[0].astype(jnp.int32), loop, jnp.zeros((pad,), jnp.int32)])
    dst = jnp.concatenate(
        [edge_index[1].astype(jnp.int32), loop,
         jnp.full((pad,), NNODE, jnp.int32)])
    xp = jnp.zeros((NPAD, DIN), jnp.float32).at[:NNODE].set(x)

    xl0, xl1, xr0, xr1 = _matmul_call(xp, W_l, W_r)
    w = _edge_logits(src, dst, xl0, xl1, xr0, xr1, att)
    acc0, acc1, accw = _aggregate(src, dst, w, xl0, xl1)
    return _finalize_call(acc0, acc1, accw, bias.reshape(1, DOUT))


# speedup vs baseline: 2.5236x; 2.5236x over previous
import functools

import jax
import jax.numpy as jnp
from jax import lax
from jax.experimental import pallas as pl
from jax.experimental.pallas import tpu as pltpu
from jax.experimental.pallas import tpu_sc as plsc

NNODE = 10000
NPAD = 10240
DIN = 480
DOUT = 256
HALF = 128
NEG = 0.2
NEDGE = 160000
EP = 172032
KB = 64
NC = 2
NS = 16
LL = 16
TILES = NC * NS
E1 = EP // TILES
B1 = E1 // KB
E2 = EP // NS
B2 = E2 // KB
RPT = NPAD // NS

_mesh = plsc.VectorSubcoreMesh(core_axis_name="c", subcore_axis_name="s")
_sc_params = pltpu.CompilerParams(needs_layout_passes=False)


@functools.partial(
    pl.kernel,
    out_type=[
        jax.ShapeDtypeStruct((EP,), jnp.float32),
        jax.ShapeDtypeStruct((TILES * NPAD,), jnp.float32),
    ],
    mesh=_mesh,
    compiler_params=_sc_params,
    scratch_types=[
        pltpu.VMEM((NPAD,), jnp.float32),
        pltpu.VMEM((KB,), jnp.int32),
        pltpu.VMEM((KB,), jnp.int32),
        pltpu.VMEM((KB, HALF), jnp.float32),
        pltpu.VMEM((KB, HALF), jnp.float32),
        pltpu.VMEM((KB, HALF), jnp.float32),
        pltpu.VMEM((KB, HALF), jnp.float32),
        pltpu.VMEM((KB, LL), jnp.float32),
        pltpu.VMEM((KB,), jnp.float32),
        pltpu.VMEM((DOUT,), jnp.float32),
    ],
)
def _edge_logits(src_hbm, dst_hbm, xl0, xl1, xr0, xr1, att_hbm,
                 w_hbm, denomp_hbm,
                 dnm, srci, dsti, a0, a1, b0, b1, tbuf, wv, attv):
    c = lax.axis_index("c")
    s = lax.axis_index("s")
    wid = s * NC + c
    pltpu.sync_copy(att_hbm, attv)
    att_chunks = [attv[pl.ds(i * LL, LL)] for i in range(DOUT // LL)]
    iota = lax.iota(jnp.int32, LL)
    lane_masks = [iota == l for l in range(LL)]

    @pl.loop(0, NPAD // LL)
    def _zd(i):
        dnm[pl.ds(i * LL, LL)] = jnp.zeros((LL,), jnp.float32)

    @pl.loop(0, B1)
    def _blk(blk):
        base = wid * E1 + blk * KB
        pltpu.sync_copy(src_hbm.at[pl.ds(base, KB)], srci)
        pltpu.sync_copy(dst_hbm.at[pl.ds(base, KB)], dsti)
        pltpu.sync_copy(xl0.at[srci], a0)
        pltpu.sync_copy(xl1.at[srci], a1)
        pltpu.sync_copy(xr0.at[dsti], b0)
        pltpu.sync_copy(xr1.at[dsti], b1)

        @pl.loop(0, KB)
        def _edge(e):
            acc = jnp.zeros((LL,), jnp.float32)
            for half, (ab, bb) in enumerate(((a0, b0), (a1, b1))):
                for q in range(HALF // LL):
                    sl = pl.ds(q * LL, LL)
                    sv = ab[e, sl] + bb[e, sl]
                    m = jnp.maximum(sv, NEG * sv)
                    acc = acc + att_chunks[half * (HALF // LL) + q] * m
            tbuf[e, :] = acc

        for g in range(KB // LL):
            rows = iota + (g * LL)
            tot = jnp.zeros((LL,), jnp.float32)
            for cc in range(LL):
                cols = jnp.full((LL,), cc, jnp.int32)
                tot = tot + plsc.load_gather(tbuf, [rows, cols])
            wv[pl.ds(g * LL, LL)] = jnp.exp(tot)

        for g in range(KB // LL):
            d16 = dsti[pl.ds(g * LL, LL)]
            w16 = wv[pl.ds(g * LL, LL)]
            for l in range(LL):
                plsc.addupdate_scatter(dnm, [d16], w16, mask=lane_masks[l])

        pltpu.sync_copy(wv, w_hbm.at[pl.ds(base, KB)])

    pltpu.sync_copy(dnm, denomp_hbm.at[pl.ds(wid * NPAD, NPAD)])


@functools.partial(
    pl.kernel,
    out_type=[
        jax.ShapeDtypeStruct((NPAD, HALF), jnp.float32),
        jax.ShapeDtypeStruct((NPAD, HALF), jnp.float32),
    ],
    mesh=_mesh,
    compiler_params=_sc_params,
    scratch_types=[
        pltpu.VMEM((KB,), jnp.int32),
        pltpu.VMEM((KB,), jnp.int32),
        pltpu.VMEM((KB,), jnp.int32),
        pltpu.VMEM((KB,), jnp.float32),
        pltpu.VMEM((KB, HALF), jnp.float32),
        pltpu.VMEM((KB, HALF), jnp.float32),
        pltpu.VMEM_SHARED((NPAD, HALF), jnp.float32),
    ],
)
def _aggregate(src_hbm, dst_hbm, w_hbm, xl0, xl1,
               acc0_hbm, acc1_hbm,
               srci, dsti, ridx, wv, rows, zb, accv):
    c = lax.axis_index("c")
    s = lax.axis_index("s")
    iota = lax.iota(jnp.int32, LL)
    rbase = s * RPT

    @pl.loop(0, KB)
    def _z(i):
        for q in range(HALF // LL):
            zb[i, pl.ds(q * LL, LL)] = jnp.zeros((LL,), jnp.float32)

    @pl.loop(0, RPT // KB)
    def _zz(i):
        b = rbase + i * KB
        for g in range(KB // LL):
            ridx[pl.ds(g * LL, LL)] = iota + (b + g * LL)
        pltpu.sync_copy(zb, accv.at[ridx])

    plsc.subcore_barrier()

    @pl.loop(0, B2)
    def _blk(blk):
        base = s * E2 + blk * KB
        pltpu.sync_copy(src_hbm.at[pl.ds(base, KB)], srci)
        pltpu.sync_copy(dst_hbm.at[pl.ds(base, KB)], dsti)
        pltpu.sync_copy(w_hbm.at[pl.ds(base, KB)], wv)

        @pl.when(c == 0)
        def _():
            pltpu.sync_copy(xl0.at[srci], rows)

        @pl.when(c == 1)
        def _():
            pltpu.sync_copy(xl1.at[srci], rows)

        @pl.loop(0, KB)
        def _edge(e):
            espl = jnp.zeros((LL,), jnp.int32) + e
            wspl = plsc.load_gather(wv, [espl])
            for q in range(HALF // LL):
                sl = pl.ds(q * LL, LL)
                rows[e, sl] = rows[e, sl] * wspl

        pltpu.sync_copy(rows, accv.at[dsti], add=True)

    plsc.subcore_barrier()

    @pl.loop(0, RPT // KB)
    def _f(i):
        b = rbase + i * KB
        for g in range(KB // LL):
            ridx[pl.ds(g * LL, LL)] = iota + (b + g * LL)
        pltpu.sync_copy(accv.at[ridx], rows)

        @pl.when(c == 0)
        def _():
            pltpu.sync_copy(rows, acc0_hbm.at[ridx])

        @pl.when(c == 1)
        def _():
            pltpu.sync_copy(rows, acc1_hbm.at[ridx])


def _matmul_call(xp, W_l, W_r):
    bm = 640

    def mm_kernel(x_ref, wl_ref, wr_ref, o0, o1, o2, o3):
        xb = x_ref[...]
        xl = jnp.dot(xb, wl_ref[...], preferred_element_type=jnp.float32,
                     precision=lax.Precision.HIGHEST)
        xr = jnp.dot(xb, wr_ref[...], preferred_element_type=jnp.float32,
                     precision=lax.Precision.HIGHEST)
        o0[...] = xl[:, :HALF]
        o1[...] = xl[:, HALF:]
        o2[...] = xr[:, :HALF]
        o3[...] = xr[:, HALF:]

    return pl.pallas_call(
        mm_kernel,
        grid=(NPAD // bm,),
        in_specs=[
            pl.BlockSpec((bm, DIN), lambda i: (i, 0)),
            pl.BlockSpec((DIN, DOUT), lambda i: (0, 0)),
            pl.BlockSpec((DIN, DOUT), lambda i: (0, 0)),
        ],
        out_specs=[pl.BlockSpec((bm, HALF), lambda i: (i, 0))] * 4,
        out_shape=[jax.ShapeDtypeStruct((NPAD, HALF), jnp.float32)] * 4,
    )(xp, W_l, W_r)


def _finalize_call(acc0, acc1, denomp2d, bias2d):
    bm = 512

    def fin_kernel(a0, a1, dp, b_ref, o_ref):
        d = jnp.sum(dp[...], axis=0)[:, None]
        o_ref[...] = jnp.concatenate([a0[...], a1[...]], axis=1) / d + b_ref[...]

    return pl.pallas_call(
        fin_kernel,
        grid=(NPAD // bm,),
        in_specs=[
            pl.BlockSpec((bm, HALF), lambda i: (i, 0)),
            pl.BlockSpec((bm, HALF), lambda i: (i, 0)),
            pl.BlockSpec((TILES, bm), lambda i: (0, i)),
            pl.BlockSpec((1, DOUT), lambda i: (0, 0)),
        ],
        out_specs=pl.BlockSpec((bm, DOUT), lambda i: (i, 0)),
        out_shape=jax.ShapeDtypeStruct((NPAD, DOUT), jnp.float32),
    )(acc0, acc1, denomp2d, bias2d)


def kernel(x, edge_index, W_l, W_r, att, bias):
    loop = jnp.arange(NNODE, dtype=jnp.int32)
    pad = EP - NEDGE - NNODE
    src = jnp.concatenate(
        [edge_index[0].astype(jnp.int32), loop, jnp.zeros((pad,), jnp.int32)])
    dst = jnp.concatenate(
        [edge_index[1].astype(jnp.int32), loop,
         jnp.full((pad,), NNODE, jnp.int32)])
    xp = jnp.zeros((NPAD, DIN), jnp.float32).at[:NNODE].set(x)

    xl0, xl1, xr0, xr1 = _matmul_call(xp, W_l, W_r)
    w, denomp = _edge_logits(src, dst, xl0, xl1, xr0, xr1, att)
    acc0, acc1 = _aggregate(src, dst, w, xl0, xl1)
    out = _finalize_call(acc0, acc1, denomp.reshape(TILES, NPAD),
                         bias.reshape(1, DOUT))
    return out[:NNODE]
